# Initial kernel scaffold; baseline (speedup 1.0000x reference)
import jax
import jax.numpy as jnp
from jax.experimental import pallas as pl


def kernel(x, edge_index, edge_weight, W1, b1, W2, b2, W3, b3):
    raise NotImplementedError("write your pallas kernel here")


# speedup vs baseline: 8.5591x; 8.5591x over previous
import functools

import jax
import jax.numpy as jnp
from jax import lax
from jax.experimental import pallas as pl
from jax.experimental.pallas import tpu as pltpu
from jax.experimental.pallas import tpu_sc as plsc

N = 10000
E = 320000
NP = 10240
EP = 331776
C_IN = 128
C_HID = 256
C_OUT = 128
NC = 2
NS = 16
NW = NC * NS
K = 128
RPT = NP // NS
RB = 512
GRID = NP // RB

_MESH = plsc.VectorSubcoreMesh(core_axis_name="c", subcore_axis_name="s")


def _deg_body(colx, wx, out, colb, wb, zb, acc, sem):
    del sem
    c = lax.axis_index("c")
    s = lax.axis_index("s")
    for j in range(RPT // 16):
        zb[pl.ds(j * 16, 16)] = jnp.zeros((16,), jnp.float32)
    pltpu.sync_copy(zb, acc.at[pl.ds(s * RPT, RPT)])
    plsc.subcore_barrier()

    epw = EP // NW
    base0 = (c * NS + s) * epw

    def blk(i, carry):
        b = base0 + i * K
        pltpu.sync_copy(colx.at[pl.ds(b, K)], colb)
        pltpu.sync_copy(wx.at[pl.ds(b, K)], wb)
        pltpu.sync_copy(wb, acc.at[colb], add=True)
        return carry

    lax.fori_loop(0, epw // K, blk, 0)
    plsc.subcore_barrier()
    pltpu.sync_copy(acc.at[pl.ds(s * RPT, RPT)], out.at[c, pl.ds(s * RPT, RPT)])


_deg_call = functools.partial(
    pl.kernel,
    out_type=jax.ShapeDtypeStruct((NC, NP), jnp.float32),
    mesh=_MESH,
    scratch_types=[
        pltpu.VMEM((K,), jnp.int32),
        pltpu.VMEM((K,), jnp.float32),
        pltpu.VMEM((RPT,), jnp.float32),
        pltpu.VMEM_SHARED((NP,), jnp.float32),
        pltpu.SemaphoreType.DMA,
    ],
)(_deg_body)


def _dis_body(degp, dis):
    d = degp[0:NP // 128, :] + degp[NP // 128:2 * (NP // 128), :]
    dis[...] = jnp.where(d > 0.0, lax.rsqrt(d), 0.0)


_dis_call = pl.pallas_call(
    _dis_body,
    out_shape=jax.ShapeDtypeStruct((NP // 128, 128), jnp.float32),
)


def _norm_body(rowx, colx, wx, dis, nrm, rowb, colb, wb, dr, dc, nb, sem):
    c = lax.axis_index("c")
    s = lax.axis_index("s")
    epw = EP // NW
    base0 = (c * NS + s) * epw

    def blk(i, carry):
        b = base0 + i * K
        pltpu.sync_copy(rowx.at[pl.ds(b, K)], rowb)
        pltpu.sync_copy(colx.at[pl.ds(b, K)], colb)
        pltpu.sync_copy(wx.at[pl.ds(b, K)], wb)
        pltpu.async_copy(dis.at[rowb], dr, sem).wait()
        pltpu.async_copy(dis.at[colb], dc, sem).wait()
        for j in range(K // 16):
            sl = pl.ds(j * 16, 16)
            nb[sl] = dr[sl] * wb[sl] * dc[sl]
        pltpu.sync_copy(nb, nrm.at[pl.ds(b, K)])
        return carry

    lax.fori_loop(0, epw // K, blk, 0)


_norm_call = functools.partial(
    pl.kernel,
    out_type=jax.ShapeDtypeStruct((EP,), jnp.float32),
    mesh=_MESH,
    scratch_types=[
        pltpu.VMEM((K,), jnp.int32),
        pltpu.VMEM((K,), jnp.int32),
        pltpu.VMEM((K,), jnp.float32),
        pltpu.VMEM((K,), jnp.float32),
        pltpu.VMEM((K,), jnp.float32),
        pltpu.VMEM((K,), jnp.float32),
        pltpu.SemaphoreType.DMA,
    ],
)(_norm_body)


def _prop_body(split_edges, srcs, rowx, colx, nrm, out,
               rowb, colb, nb, xrows, zb, acc, sem):
    c = lax.axis_index("c")
    s = lax.axis_index("s")
    def zrow(r, carry):
        for j in range(C_IN // 16):
            zb[r, pl.ds(j * 16, 16)] = jnp.zeros((16,), jnp.float32)
        return carry
    lax.fori_loop(0, 64, zrow, 0)

    def zcp(t, carry):
        pltpu.sync_copy(zb, acc.at[pl.ds(s * RPT + t * 64, 64)])
        return carry
    lax.fori_loop(0, RPT // 64, zcp, 0)
    plsc.subcore_barrier()

    if split_edges:
        epw = EP // NW
        base0 = (c * NS + s) * epw
    else:
        epw = EP // NS
        base0 = s * epw

    def blk(i, carry):
        b = base0 + i * K
        pltpu.sync_copy(rowx.at[pl.ds(b, K)], rowb)
        pltpu.sync_copy(colx.at[pl.ds(b, K)], colb)
        pltpu.sync_copy(nrm.at[pl.ds(b, K)], nb)
        if len(srcs) == 1:
            pltpu.async_copy(srcs[0].at[rowb], xrows, sem).wait()
        else:
            @pl.when(c == 0)
            def _():
                pltpu.async_copy(srcs[0].at[rowb], xrows, sem).wait()

            @pl.when(c == 1)
            def _():
                pltpu.async_copy(srcs[1].at[rowb], xrows, sem).wait()

        def grp(g, carry2):
            wv = nb[pl.ds(g * 16, 16)]
            for t in range(16):
                w = wv[t]
                k = g * 16 + t
                for j in range(C_IN // 16):
                    sl = pl.ds(j * 16, 16)
                    xrows[k, sl] = xrows[k, sl] * w
            return carry2

        lax.fori_loop(0, K // 16, grp, 0)
        pltpu.sync_copy(xrows, acc.at[colb], add=True)
        return carry

    lax.fori_loop(0, epw // K, blk, 0)
    plsc.subcore_barrier()
    pltpu.sync_copy(acc.at[pl.ds(s * RPT, RPT)],
                    out.at[c, pl.ds(s * RPT, RPT)])


def _make_prop(split_edges, n_src):
    def body(*refs):
        srcs = refs[:n_src]
        rest = refs[n_src:]
        _prop_body(split_edges, srcs, *rest)

    return functools.partial(
        pl.kernel,
        out_type=jax.ShapeDtypeStruct((NC, NP, C_IN), jnp.float32),
        mesh=_MESH,
        scratch_types=[
            pltpu.VMEM((K,), jnp.int32),
            pltpu.VMEM((K,), jnp.int32),
            pltpu.VMEM((K,), jnp.float32),
            pltpu.VMEM((K, C_IN), jnp.float32),
            pltpu.VMEM((64, C_IN), jnp.float32),
            pltpu.VMEM_SHARED((NP, C_IN), jnp.float32),
            pltpu.SemaphoreType.DMA,
        ],
    )(body)


_prop1_call = _make_prop(split_edges=True, n_src=1)
_prop2_call = _make_prop(split_edges=False, n_src=2)


def _mid_body(s1, w1, b1, outa, outb):
    acc = s1[0] + s1[1]
    h = jnp.dot(acc, w1[...], preferred_element_type=jnp.float32) + b1[...]
    h = jnp.maximum(h, 0.0)
    outa[...] = h[:, :C_IN]
    outb[...] = h[:, C_IN:]


_mid_call = pl.pallas_call(
    _mid_body,
    grid=(GRID,),
    in_specs=[
        pl.BlockSpec((NC, RB, C_IN), lambda i: (0, i, 0)),
        pl.BlockSpec((C_IN, C_HID), lambda i: (0, 0)),
        pl.BlockSpec((1, C_HID), lambda i: (0, 0)),
    ],
    out_specs=[
        pl.BlockSpec((RB, C_IN), lambda i: (i, 0)),
        pl.BlockSpec((RB, C_IN), lambda i: (i, 0)),
    ],
    out_shape=[
        jax.ShapeDtypeStruct((NP, C_IN), jnp.float32),
        jax.ShapeDtypeStruct((NP, C_IN), jnp.float32),
    ],
)


def _fin_body(s2, w2, w3, b2, b3, mean, logstd):
    h = jnp.concatenate([s2[0], s2[1]], axis=1)
    mean[...] = jnp.dot(h, w2[...], preferred_element_type=jnp.float32) + b2[...]
    logstd[...] = jnp.dot(h, w3[...], preferred_element_type=jnp.float32) + b3[...]


_fin_call = pl.pallas_call(
    _fin_body,
    grid=(GRID,),
    in_specs=[
        pl.BlockSpec((NC, RB, C_IN), lambda i: (0, i, 0)),
        pl.BlockSpec((C_HID, C_OUT), lambda i: (0, 0)),
        pl.BlockSpec((C_HID, C_OUT), lambda i: (0, 0)),
        pl.BlockSpec((1, C_OUT), lambda i: (0, 0)),
        pl.BlockSpec((1, C_OUT), lambda i: (0, 0)),
    ],
    out_specs=[
        pl.BlockSpec((RB, C_OUT), lambda i: (i, 0)),
        pl.BlockSpec((RB, C_OUT), lambda i: (i, 0)),
    ],
    out_shape=[
        jax.ShapeDtypeStruct((NP, C_OUT), jnp.float32),
        jax.ShapeDtypeStruct((NP, C_OUT), jnp.float32),
    ],
)


@jax.jit
def kernel(x, edge_index, edge_weight, W1, b1, W2, b2, W3, b3):
    row = edge_index[0].astype(jnp.int32)
    col = edge_index[1].astype(jnp.int32)
    loop = jnp.arange(NP, dtype=jnp.int32)
    npad = EP - E - NP
    rowx = jnp.concatenate([row, loop, jnp.zeros((npad,), jnp.int32)])
    colx = jnp.concatenate([col, loop, jnp.zeros((npad,), jnp.int32)])
    wx = jnp.concatenate([edge_weight.astype(jnp.float32),
                          jnp.ones((NP,), jnp.float32),
                          jnp.zeros((npad,), jnp.float32)])
    xp = jnp.pad(x.astype(jnp.float32), ((0, NP - N), (0, 0)))

    degp = _deg_call(colx, wx)
    dis2d = _dis_call(degp.reshape(2 * (NP // 128), 128))
    dis = dis2d.reshape(NP)
    nrm = _norm_call(rowx, colx, wx, dis)
    s1 = _prop1_call(xp, rowx, colx, nrm)
    h1a, h1b = _mid_call(s1, W1, b1.reshape(1, C_HID))
    s2 = _prop2_call(h1a, h1b, rowx, colx, nrm)
    mean, logstd = _fin_call(s2, W2, W3,
                             b2.reshape(1, C_OUT), b3.reshape(1, C_OUT))
    return mean[:N], logstd[:N]


# speedup vs baseline: 10.0541x; 1.1747x over previous
import functools

import jax
import jax.numpy as jnp
from jax import lax
from jax.experimental import pallas as pl
from jax.experimental.pallas import tpu as pltpu
from jax.experimental.pallas import tpu_sc as plsc

N = 10000
E = 320000
NP = 10240
EP = 331776
C_IN = 128
C_HID = 256
C_OUT = 128
NC = 2
NS = 16
NW = NC * NS
K = 128
RPT = NP // NS
RB = 512
GRID = NP // RB

_MESH = plsc.VectorSubcoreMesh(core_axis_name="c", subcore_axis_name="s")


def _deg_body(colx, wx, out, colb, wb, zb, acc, sem):
    del sem
    c = lax.axis_index("c")
    s = lax.axis_index("s")
    for j in range(RPT // 16):
        zb[pl.ds(j * 16, 16)] = jnp.zeros((16,), jnp.float32)
    pltpu.sync_copy(zb, acc.at[pl.ds(s * RPT, RPT)])
    plsc.subcore_barrier()

    epw = EP // NW
    base0 = (c * NS + s) * epw

    def blk(i, carry):
        b = base0 + i * K
        pltpu.sync_copy(colx.at[pl.ds(b, K)], colb)
        pltpu.sync_copy(wx.at[pl.ds(b, K)], wb)
        pltpu.sync_copy(wb, acc.at[colb], add=True)
        return carry

    lax.fori_loop(0, epw // K, blk, 0)
    plsc.subcore_barrier()
    pltpu.sync_copy(acc.at[pl.ds(s * RPT, RPT)], out.at[c, pl.ds(s * RPT, RPT)])


_deg_call = functools.partial(
    pl.kernel,
    out_type=jax.ShapeDtypeStruct((NC, NP), jnp.float32),
    mesh=_MESH,
    scratch_types=[
        pltpu.VMEM((K,), jnp.int32),
        pltpu.VMEM((K,), jnp.float32),
        pltpu.VMEM((RPT,), jnp.float32),
        pltpu.VMEM_SHARED((NP,), jnp.float32),
        pltpu.SemaphoreType.DMA,
    ],
)(_deg_body)


def _dis_body(degp, dis):
    d = degp[0:NP // 128, :] + degp[NP // 128:2 * (NP // 128), :]
    dis[...] = jnp.where(d > 0.0, lax.rsqrt(d), 0.0)


_dis_call = pl.pallas_call(
    _dis_body,
    out_shape=jax.ShapeDtypeStruct((NP // 128, 128), jnp.float32),
)


def _prop1_body(src, rowx, colx, wx, dis, out, nrm,
                rowb, colb, nb, wb, dr, dc, xrows, zb, acc, sem):
    c = lax.axis_index("c")
    s = lax.axis_index("s")
    def zrow(r, carry):
        for j in range(C_IN // 16):
            zb[r, pl.ds(j * 16, 16)] = jnp.zeros((16,), jnp.float32)
        return carry
    lax.fori_loop(0, 64, zrow, 0)

    def zcp(t, carry):
        pltpu.sync_copy(zb, acc.at[pl.ds(s * RPT + t * 64, 64)])
        return carry
    lax.fori_loop(0, RPT // 64, zcp, 0)
    plsc.subcore_barrier()

    epw = EP // NW
    base0 = (c * NS + s) * epw

    def blk(i, carry):
        b = base0 + i * K
        pltpu.sync_copy(rowx.at[pl.ds(b, K)], rowb)
        pltpu.sync_copy(colx.at[pl.ds(b, K)], colb)
        pltpu.sync_copy(wx.at[pl.ds(b, K)], wb)
        cp_r = pltpu.async_copy(dis.at[rowb], dr, sem)
        cp_c = pltpu.async_copy(dis.at[colb], dc, sem)
        cp_x = pltpu.async_copy(src.at[rowb], xrows, sem)
        cp_r.wait()
        cp_c.wait()
        for j in range(K // 16):
            sl = pl.ds(j * 16, 16)
            nb[sl] = dr[sl] * wb[sl] * dc[sl]
        pltpu.sync_copy(nb, nrm.at[pl.ds(b, K)])
        cp_x.wait()

        def grp(g, carry2):
            wv = nb[pl.ds(g * 16, 16)]
            for t in range(16):
                w = wv[t]
                k = g * 16 + t
                for j in range(C_IN // 16):
                    sl = pl.ds(j * 16, 16)
                    xrows[k, sl] = xrows[k, sl] * w
            return carry2

        lax.fori_loop(0, K // 16, grp, 0)
        pltpu.sync_copy(xrows, acc.at[colb], add=True)
        return carry

    lax.fori_loop(0, epw // K, blk, 0)
    plsc.subcore_barrier()
    pltpu.sync_copy(acc.at[pl.ds(s * RPT, RPT)],
                    out.at[c, pl.ds(s * RPT, RPT)])


_prop1_call = functools.partial(
    pl.kernel,
    out_type=(jax.ShapeDtypeStruct((NC, NP, C_IN), jnp.float32),
              jax.ShapeDtypeStruct((EP,), jnp.float32)),
    mesh=_MESH,
    scratch_types=[
        pltpu.VMEM((K,), jnp.int32),
        pltpu.VMEM((K,), jnp.int32),
        pltpu.VMEM((K,), jnp.float32),
        pltpu.VMEM((K,), jnp.float32),
        pltpu.VMEM((K,), jnp.float32),
        pltpu.VMEM((K,), jnp.float32),
        pltpu.VMEM((K, C_IN), jnp.float32),
        pltpu.VMEM((64, C_IN), jnp.float32),
        pltpu.VMEM_SHARED((NP, C_IN), jnp.float32),
        pltpu.SemaphoreType.DMA,
    ],
)(_prop1_body)


def _prop_body(split_edges, srcs, rowx, colx, nrm, out,
               rowb, colb, nb, xrows, zb, acc, sem):
    c = lax.axis_index("c")
    s = lax.axis_index("s")
    def zrow(r, carry):
        for j in range(C_IN // 16):
            zb[r, pl.ds(j * 16, 16)] = jnp.zeros((16,), jnp.float32)
        return carry
    lax.fori_loop(0, 64, zrow, 0)

    def zcp(t, carry):
        pltpu.sync_copy(zb, acc.at[pl.ds(s * RPT + t * 64, 64)])
        return carry
    lax.fori_loop(0, RPT // 64, zcp, 0)
    plsc.subcore_barrier()

    if split_edges:
        epw = EP // NW
        base0 = (c * NS + s) * epw
    else:
        epw = EP // NS
        base0 = s * epw

    def blk(i, carry):
        b = base0 + i * K
        pltpu.sync_copy(rowx.at[pl.ds(b, K)], rowb)
        pltpu.sync_copy(colx.at[pl.ds(b, K)], colb)
        pltpu.sync_copy(nrm.at[pl.ds(b, K)], nb)
        if len(srcs) == 1:
            pltpu.async_copy(srcs[0].at[rowb], xrows, sem).wait()
        else:
            @pl.when(c == 0)
            def _():
                pltpu.async_copy(srcs[0].at[rowb], xrows, sem).wait()

            @pl.when(c == 1)
            def _():
                pltpu.async_copy(srcs[1].at[rowb], xrows, sem).wait()

        def grp(g, carry2):
            wv = nb[pl.ds(g * 16, 16)]
            for t in range(16):
                w = wv[t]
                k = g * 16 + t
                for j in range(C_IN // 16):
                    sl = pl.ds(j * 16, 16)
                    xrows[k, sl] = xrows[k, sl] * w
            return carry2

        lax.fori_loop(0, K // 16, grp, 0)
        pltpu.sync_copy(xrows, acc.at[colb], add=True)
        return carry

    lax.fori_loop(0, epw // K, blk, 0)
    plsc.subcore_barrier()
    pltpu.sync_copy(acc.at[pl.ds(s * RPT, RPT)],
                    out.at[c, pl.ds(s * RPT, RPT)])


def _make_prop(split_edges, n_src):
    def body(*refs):
        srcs = refs[:n_src]
        rest = refs[n_src:]
        _prop_body(split_edges, srcs, *rest)

    return functools.partial(
        pl.kernel,
        out_type=jax.ShapeDtypeStruct((NC, NP, C_IN), jnp.float32),
        mesh=_MESH,
        scratch_types=[
            pltpu.VMEM((K,), jnp.int32),
            pltpu.VMEM((K,), jnp.int32),
            pltpu.VMEM((K,), jnp.float32),
            pltpu.VMEM((K, C_IN), jnp.float32),
            pltpu.VMEM((64, C_IN), jnp.float32),
            pltpu.VMEM_SHARED((NP, C_IN), jnp.float32),
            pltpu.SemaphoreType.DMA,
        ],
    )(body)


_prop2_call = _make_prop(split_edges=False, n_src=2)


def _mid_body(s1, w1, b1, outa, outb):
    acc = s1[0] + s1[1]
    h = jnp.dot(acc, w1[...], preferred_element_type=jnp.float32) + b1[...]
    h = jnp.maximum(h, 0.0)
    outa[...] = h[:, :C_IN]
    outb[...] = h[:, C_IN:]


_mid_call = pl.pallas_call(
    _mid_body,
    grid=(GRID,),
    in_specs=[
        pl.BlockSpec((NC, RB, C_IN), lambda i: (0, i, 0)),
        pl.BlockSpec((C_IN, C_HID), lambda i: (0, 0)),
        pl.BlockSpec((1, C_HID), lambda i: (0, 0)),
    ],
    out_specs=[
        pl.BlockSpec((RB, C_IN), lambda i: (i, 0)),
        pl.BlockSpec((RB, C_IN), lambda i: (i, 0)),
    ],
    out_shape=[
        jax.ShapeDtypeStruct((NP, C_IN), jnp.float32),
        jax.ShapeDtypeStruct((NP, C_IN), jnp.float32),
    ],
)


def _fin_body(s2, w2, w3, b2, b3, mean, logstd):
    h = jnp.concatenate([s2[0], s2[1]], axis=1)
    mean[...] = jnp.dot(h, w2[...], preferred_element_type=jnp.float32) + b2[...]
    logstd[...] = jnp.dot(h, w3[...], preferred_element_type=jnp.float32) + b3[...]


_fin_call = pl.pallas_call(
    _fin_body,
    grid=(GRID,),
    in_specs=[
        pl.BlockSpec((NC, RB, C_IN), lambda i: (0, i, 0)),
        pl.BlockSpec((C_HID, C_OUT), lambda i: (0, 0)),
        pl.BlockSpec((C_HID, C_OUT), lambda i: (0, 0)),
        pl.BlockSpec((1, C_OUT), lambda i: (0, 0)),
        pl.BlockSpec((1, C_OUT), lambda i: (0, 0)),
    ],
    out_specs=[
        pl.BlockSpec((RB, C_OUT), lambda i: (i, 0)),
        pl.BlockSpec((RB, C_OUT), lambda i: (i, 0)),
    ],
    out_shape=[
        jax.ShapeDtypeStruct((NP, C_OUT), jnp.float32),
        jax.ShapeDtypeStruct((NP, C_OUT), jnp.float32),
    ],
)


@jax.jit
def kernel(x, edge_index, edge_weight, W1, b1, W2, b2, W3, b3):
    row = edge_index[0].astype(jnp.int32)
    col = edge_index[1].astype(jnp.int32)
    loop = jnp.arange(NP, dtype=jnp.int32)
    npad = EP - E - NP
    rowx = jnp.concatenate([row, loop, jnp.zeros((npad,), jnp.int32)])
    colx = jnp.concatenate([col, loop, jnp.zeros((npad,), jnp.int32)])
    wx = jnp.concatenate([edge_weight.astype(jnp.float32),
                          jnp.ones((NP,), jnp.float32),
                          jnp.zeros((npad,), jnp.float32)])
    xp = jnp.pad(x.astype(jnp.float32), ((0, NP - N), (0, 0)))

    degp = _deg_call(colx, wx)
    dis2d = _dis_call(degp.reshape(2 * (NP // 128), 128))
    dis = dis2d.reshape(NP)
    s1, nrm = _prop1_call(xp, rowx, colx, wx, dis)
    h1a, h1b = _mid_call(s1, W1, b1.reshape(1, C_HID))
    s2 = _prop2_call(h1a, h1b, rowx, colx, nrm)
    mean, logstd = _fin_call(s2, W2, W3,
                             b2.reshape(1, C_OUT), b3.reshape(1, C_OUT))
    return mean[:N], logstd[:N]
